# baseline (speedup 1.0000x reference)
import functools

import jax
import jax.numpy as jnp
from jax import lax
from jax.experimental import pallas as pl
from jax.experimental.pallas import tpu as pltpu
from jax.experimental.pallas import tpu_sc as plsc

N = 10000
NPAD = 10240
E = 320000
D = 128
NC = 2
NS = 16
NW = NC * NS
EPAD = NW * 10240
EPT = EPAD // NW
CH = 64
NCHT = EPAD // (NS * CH)
NCH0 = 256
NCH1 = NCHT - NCH0
ROWS_PT = NPAD // NW
ROWS_PC = NPAD // NS

_HIGH = jax.lax.Precision.HIGHEST


def _mm_body(x_ref, w_ref, a_ref, f_ref, elr_ref):
    f = jnp.dot(x_ref[...], w_ref[...], precision=_HIGH,
                preferred_element_type=jnp.float32)
    f_ref[...] = f
    elr_ref[...] = jnp.dot(f, a_ref[...], precision=_HIGH,
                           preferred_element_type=jnp.float32)


def _mm(x, w, acols, blk=1024):
    grid = (NPAD // blk,)
    return pl.pallas_call(
        _mm_body,
        grid=grid,
        in_specs=[
            pl.BlockSpec((blk, D), lambda i: (i, 0)),
            pl.BlockSpec((D, D), lambda i: (0, 0)),
            pl.BlockSpec((D, D), lambda i: (0, 0)),
        ],
        out_specs=[
            pl.BlockSpec((blk, D), lambda i: (i, 0)),
            pl.BlockSpec((blk, D), lambda i: (i, 0)),
        ],
        out_shape=[
            jax.ShapeDtypeStruct((NPAD, D), jnp.float32),
            jax.ShapeDtypeStruct((NPAD, D), jnp.float32),
        ],
    )(x, w, acols)


def _head_body(h_ref, fcw_ref, fcb_ref, o_ref):
    s = jnp.sum(h_ref[...], axis=0, keepdims=True) * (1.0 / N)
    sb = jnp.broadcast_to(s, (8, D))
    o_ref[...] = jnp.dot(sb, fcw_ref[...], precision=_HIGH,
                         preferred_element_type=jnp.float32) + fcb_ref[...]


def _head(h, fcw_pad, fcb_pad):
    return pl.pallas_call(
        _head_body,
        in_specs=[
            pl.BlockSpec((NPAD, D), lambda: (0, 0)),
            pl.BlockSpec((D, D), lambda: (0, 0)),
            pl.BlockSpec((8, D), lambda: (0, 0)),
        ],
        out_specs=pl.BlockSpec((8, D), lambda: (0, 0)),
        out_shape=jax.ShapeDtypeStruct((8, D), jnp.float32),
    )(h, fcw_pad, fcb_pad)


def _edge_body(feat_hbm, el_hbm, er_hbm, src_hbm, dst_hbm,
               outp_hbm, denp_hbm,
               el_v, er_v, den_v, srcc_v, dstc_v, eec_v, rows_v, out_sh,
               isem0, isem1, isem2, isem3, gsem0, gsem1):
    c = lax.axis_index("c")
    s = lax.axis_index("s")
    wid = s * NC + c

    pltpu.sync_copy(el_hbm, el_v)
    pltpu.sync_copy(er_hbm, er_v)

    zv = jnp.zeros((16,), jnp.float32)

    def zden(i, carry):
        den_v[pl.ds(pl.multiple_of(i * 16, 16), 16)] = zv
        return carry
    lax.fori_loop(0, NPAD // 16, zden, 0)

    def zrow(i, carry):
        for b in range(2):
            for j in range(8):
                rows_v[b, i, pl.ds(16 * j, 16)] = zv
        return carry
    lax.fori_loop(0, CH, zrow, 0)

    def zout(z, carry):
        pltpu.sync_copy(rows_v.at[0],
                        out_sh.at[pl.ds(s * ROWS_PC + z * CH, CH)])
        return carry
    lax.fori_loop(0, ROWS_PC // CH, zout, 0)

    plsc.subcore_barrier()

    def fire_idx(ci, b, isem):
        pltpu.async_copy(src_hbm.at[s, ci], srcc_v.at[b], isem)
        pltpu.async_copy(dst_hbm.at[s, ci], dstc_v.at[b], isem)

    def wait_idx(ci, b, isem):
        pltpu.make_async_copy(src_hbm.at[s, ci], srcc_v.at[b], isem).wait()
        pltpu.make_async_copy(dst_hbm.at[s, ci], dstc_v.at[b], isem).wait()

    def fire_g(k, gsem):
        pltpu.async_copy(feat_hbm.at[srcc_v.at[k, 0]], rows_v.at[k % 2], gsem)

    def process(k, gsem):
        b = k % 2
        for kk in range(CH // 16):
            sl = pl.ds(16 * kk, 16)
            si = srcc_v[k, 0, sl]
            di = dstc_v[k, 0, sl]
            e = plsc.load_gather(el_v, [si]) + plsc.load_gather(er_v, [di])
            e = jnp.where(e > 0, e, 0.2 * e)
            ee = jnp.exp(e)
            eec_v[b, sl] = ee
            plsc.addupdate_scatter(den_v, [di], ee)

        pltpu.make_async_copy(feat_hbm.at[srcc_v.at[k, 0]], rows_v.at[b],
                              gsem).wait()

        def grp(g, rc):
            a16 = eec_v[b, pl.ds(pl.multiple_of(g * 16, 16), 16)]
            for r16 in range(16):
                r = g * 16 + r16
                a = a16[r16]
                for j in range(8):
                    sl = pl.ds(16 * j, 16)
                    rows_v[b, r, sl] = rows_v[b, r, sl] * a
            return rc
        lax.fori_loop(0, CH // 16, grp, 0)
        pltpu.sync_copy(rows_v.at[b], out_sh.at[dstc_v.at[k, 0]], add=True)

    cstart = jnp.where(c == 0, 0, NCH0)
    NQ = jnp.where(c == 0, NCH0 // 4, NCH1 // 4)
    isems = (isem0, isem1, isem2, isem3)
    gsems = (gsem0, gsem1)

    def AB(ci, k):
        wait_idx(ci, k, isems[k])
        fire_g(k, gsems[k % 2])

    for k in range(4):
        fire_idx(cstart + k, k, isems[k])
    AB(cstart, 0)
    AB(cstart + 1, 1)

    def quad(t, carry):
        i0 = cstart + 4 * t
        inr = t + 1 < NQ

        process(0, gsems[0])

        @pl.when(inr)
        def _():
            fire_idx(i0 + 4, 0, isems[0])
        AB(i0 + 2, 2)
        process(1, gsems[1])

        @pl.when(inr)
        def _():
            fire_idx(i0 + 5, 1, isems[1])
        AB(i0 + 3, 3)
        process(2, gsems[0])

        @pl.when(inr)
        def _():
            fire_idx(i0 + 6, 2, isems[2])
            AB(i0 + 4, 0)
        process(3, gsems[1])

        @pl.when(inr)
        def _():
            fire_idx(i0 + 7, 3, isems[3])
            AB(i0 + 5, 1)
        return carry
    lax.fori_loop(0, NQ, quad, 0)

    pltpu.sync_copy(den_v, denp_hbm.at[wid])

    plsc.subcore_barrier()
    pltpu.sync_copy(out_sh.at[pl.ds(s * ROWS_PC, ROWS_PC)],
                    outp_hbm.at[c, pl.ds(s * ROWS_PC, ROWS_PC)])


def _edge_pass(feat, el, er, src3, dst3):
    mesh = plsc.VectorSubcoreMesh(core_axis_name="c", subcore_axis_name="s",
                                  num_cores=NC, num_subcores=NS)
    fn = pl.kernel(
        _edge_body,
        compiler_params=pltpu.CompilerParams(needs_layout_passes=False),
        out_type=[
            jax.ShapeDtypeStruct((NC, NPAD, D), jnp.float32),
            jax.ShapeDtypeStruct((NW, NPAD), jnp.float32),
        ],
        mesh=mesh,
        scratch_types=[
            pltpu.VMEM((NPAD,), jnp.float32),
            pltpu.VMEM((NPAD,), jnp.float32),
            pltpu.VMEM((NPAD,), jnp.float32),
            pltpu.VMEM((4, 1, CH), jnp.int32),
            pltpu.VMEM((4, 1, CH), jnp.int32),
            pltpu.VMEM((2, CH), jnp.float32),
            pltpu.VMEM((2, CH, D), jnp.float32),
            pltpu.MemorySpace.VMEM_SHARED((NPAD, D), jnp.float32),
            pltpu.SemaphoreType.DMA,
            pltpu.SemaphoreType.DMA,
            pltpu.SemaphoreType.DMA,
            pltpu.SemaphoreType.DMA,
            pltpu.SemaphoreType.DMA,
            pltpu.SemaphoreType.DMA,
        ],
    )
    return fn(feat, el, er, src3, dst3)


def _scale_body(outp_hbm, denp_hbm, b_hbm, h_hbm,
                p0_v, p1_v, dd_v, acc_v, b_v):
    c = lax.axis_index("c")
    s = lax.axis_index("s")
    wid = s * NC + c
    R = 128
    NBLK = NPAD // R

    pltpu.sync_copy(b_hbm, b_v)

    def do_block(bid):
        base = pl.multiple_of(bid * R, R)
        bslice = pl.ds(base, R)
        pltpu.sync_copy(outp_hbm.at[0, bslice], p0_v)
        pltpu.sync_copy(outp_hbm.at[1, bslice], p1_v)
        pltpu.sync_copy(denp_hbm.at[:, bslice], dd_v)

        for j in range(R // 16):
            sl = pl.ds(16 * j, 16)
            t = dd_v[0, sl]
            for w in range(1, NW):
                t = t + dd_v[w, sl]
            acc_v[sl] = 1.0 / (t + 1e-9)

        def rgrp(g, rc):
            sc16 = acc_v[pl.ds(pl.multiple_of(g * 16, 16), 16)]
            for r16 in range(16):
                r = g * 16 + r16
                inb = (base + r) < N
                scz = jnp.where(inb, sc16[r16], 0.0)
                for j in range(8):
                    sl = pl.ds(16 * j, 16)
                    hv = jnp.maximum((p0_v[r, sl] + p1_v[r, sl]) * scz
                                     + b_v[pl.ds(16 * j, 16)], 0.0)
                    p0_v[r, sl] = jnp.where(inb, hv, 0.0)
            return rc
        lax.fori_loop(0, R // 16, rgrp, 0)
        pltpu.sync_copy(p0_v, h_hbm.at[bslice])

    do_block(wid)
    do_block(wid + NW)

    @pl.when(wid + 2 * NW < NBLK)
    def _():
        do_block(wid + 2 * NW)


def _scale_pass(outp, denp, brow):
    mesh = plsc.VectorSubcoreMesh(core_axis_name="c", subcore_axis_name="s",
                                  num_cores=NC, num_subcores=NS)
    fn = pl.kernel(
        _scale_body,
        compiler_params=pltpu.CompilerParams(needs_layout_passes=False),
        out_type=jax.ShapeDtypeStruct((NPAD, D), jnp.float32),
        mesh=mesh,
        scratch_types=[
            pltpu.VMEM((128, D), jnp.float32),
            pltpu.VMEM((128, D), jnp.float32),
            pltpu.VMEM((NW, 128), jnp.float32),
            pltpu.VMEM((128,), jnp.float32),
            pltpu.VMEM((D,), jnp.float32),
        ],
    )
    return fn(outp, denp, brow)


def kernel(x, edge_index, W1, al1, ar1, b1, W2, al2, ar2, b2, fcW, fcb):
    f32 = jnp.float32
    x_pad = jnp.zeros((NPAD, D), f32).at[:N].set(x)

    src = edge_index[0]
    dst = edge_index[1]
    pad_e = EPAD - E
    src3 = jnp.concatenate([src, jnp.zeros((pad_e,), jnp.int32)]
                           ).reshape(NS, NCHT, 1, CH)
    dst3 = jnp.concatenate([dst, jnp.full((pad_e,), NPAD - 1, jnp.int32)]
                           ).reshape(NS, NCHT, 1, CH)

    def acols(al, ar):
        a = jnp.zeros((D, D), f32)
        a = a.at[:, 0].set(al[0])
        a = a.at[:, 1].set(ar[0])
        return a

    feat1, elr1 = _mm(x_pad, W1, acols(al1, ar1))
    el1 = elr1[:, 0]
    er1 = elr1[:, 1]
    outp1, denp1 = _edge_pass(feat1, el1, er1, src3, dst3)
    h1 = _scale_pass(outp1, denp1, b1.reshape(D))

    feat2, elr2 = _mm(h1, W2, acols(al2, ar2))
    el2 = elr2[:, 0]
    er2 = elr2[:, 1]
    outp2, denp2 = _edge_pass(feat2, el2, er2, src3, dst3)
    h2 = _scale_pass(outp2, denp2, b2.reshape(D))

    fcw_pad = jnp.zeros((D, D), f32).at[:, :1].set(fcW)
    fcb_pad = jnp.broadcast_to(fcb.reshape(1, 1), (8, D)).astype(f32)
    res = _head(h2, fcw_pad, fcb_pad)
    return res[0:1, 0:1]


# speedup vs baseline: 1.0545x; 1.0545x over previous
import functools

import jax
import jax.numpy as jnp
from jax import lax
from jax.experimental import pallas as pl
from jax.experimental.pallas import tpu as pltpu
from jax.experimental.pallas import tpu_sc as plsc

N = 10000
NPAD = 10240
E = 320000
D = 128
NC = 2
NS = 16
NW = NC * NS
EPAD = NW * 10240
EPT = EPAD // NW
CH = 64
NCHT = EPAD // (NS * CH)
NCH0 = 240
NCH1 = NCHT - NCH0
ROWS_PT = NPAD // NW
ROWS_PC = NPAD // NS

_HIGH = jax.lax.Precision.HIGHEST


def _mm_body(x_ref, w_ref, a_ref, f_ref, elr_ref):
    f = jnp.dot(x_ref[...], w_ref[...], precision=_HIGH,
                preferred_element_type=jnp.float32)
    f_ref[...] = f
    elr_ref[...] = jnp.dot(f, a_ref[...], precision=_HIGH,
                           preferred_element_type=jnp.float32)


def _mm(x, w, acols, blk=1024):
    grid = (NPAD // blk,)
    return pl.pallas_call(
        _mm_body,
        grid=grid,
        in_specs=[
            pl.BlockSpec((blk, D), lambda i: (i, 0)),
            pl.BlockSpec((D, D), lambda i: (0, 0)),
            pl.BlockSpec((D, D), lambda i: (0, 0)),
        ],
        out_specs=[
            pl.BlockSpec((blk, D), lambda i: (i, 0)),
            pl.BlockSpec((blk, D), lambda i: (i, 0)),
        ],
        out_shape=[
            jax.ShapeDtypeStruct((NPAD, D), jnp.float32),
            jax.ShapeDtypeStruct((NPAD, D), jnp.float32),
        ],
    )(x, w, acols)


def _head_body(h_ref, fcw_ref, fcb_ref, o_ref):
    s = jnp.sum(h_ref[...], axis=0, keepdims=True) * (1.0 / N)
    sb = jnp.broadcast_to(s, (8, D))
    o_ref[...] = jnp.dot(sb, fcw_ref[...], precision=_HIGH,
                         preferred_element_type=jnp.float32) + fcb_ref[...]


def _head(h, fcw_pad, fcb_pad):
    return pl.pallas_call(
        _head_body,
        in_specs=[
            pl.BlockSpec((NPAD, D), lambda: (0, 0)),
            pl.BlockSpec((D, D), lambda: (0, 0)),
            pl.BlockSpec((8, D), lambda: (0, 0)),
        ],
        out_specs=pl.BlockSpec((8, D), lambda: (0, 0)),
        out_shape=jax.ShapeDtypeStruct((8, D), jnp.float32),
    )(h, fcw_pad, fcb_pad)


def _edge_body(feat_hbm, el_hbm, er_hbm, src_hbm, dst_hbm,
               outp_hbm, denp_hbm,
               el_v, er_v, den_v, srcc_v, dstc_v, eec_v, rows_v, out_sh,
               isem0, isem1, isem2, isem3, gsem0, gsem1):
    c = lax.axis_index("c")
    s = lax.axis_index("s")
    wid = s * NC + c

    pltpu.sync_copy(el_hbm, el_v)
    pltpu.sync_copy(er_hbm, er_v)

    zv = jnp.zeros((16,), jnp.float32)

    def zden(i, carry):
        den_v[pl.ds(pl.multiple_of(i * 16, 16), 16)] = zv
        return carry
    lax.fori_loop(0, NPAD // 16, zden, 0)

    def zrow(i, carry):
        for b in range(2):
            for j in range(8):
                rows_v[b, i, pl.ds(16 * j, 16)] = zv
        return carry
    lax.fori_loop(0, CH, zrow, 0)

    def zout(z, carry):
        pltpu.sync_copy(rows_v.at[0],
                        out_sh.at[pl.ds(s * ROWS_PC + z * CH, CH)])
        return carry
    lax.fori_loop(0, ROWS_PC // CH, zout, 0)

    plsc.subcore_barrier()

    def fire_idx(ci, b, isem):
        pltpu.async_copy(src_hbm.at[s, ci], srcc_v.at[b], isem)
        pltpu.async_copy(dst_hbm.at[s, ci], dstc_v.at[b], isem)

    def wait_idx(ci, b, isem):
        pltpu.make_async_copy(src_hbm.at[s, ci], srcc_v.at[b], isem).wait()
        pltpu.make_async_copy(dst_hbm.at[s, ci], dstc_v.at[b], isem).wait()

    def fire_g(k, gsem):
        pltpu.async_copy(feat_hbm.at[srcc_v.at[k, 0]], rows_v.at[k % 2], gsem)

    def process(k, gsem):
        b = k % 2
        for kk in range(CH // 16):
            sl = pl.ds(16 * kk, 16)
            si = srcc_v[k, 0, sl]
            di = dstc_v[k, 0, sl]
            e = plsc.load_gather(el_v, [si]) + plsc.load_gather(er_v, [di])
            e = jnp.where(e > 0, e, 0.2 * e)
            ee = jnp.exp(e)
            eec_v[b, sl] = ee
            plsc.addupdate_scatter(den_v, [di], ee)

        pltpu.make_async_copy(feat_hbm.at[srcc_v.at[k, 0]], rows_v.at[b],
                              gsem).wait()

        def grp(g, rc):
            a16 = eec_v[b, pl.ds(pl.multiple_of(g * 16, 16), 16)]
            for r16 in range(16):
                r = g * 16 + r16
                a = a16[r16]
                for j in range(8):
                    sl = pl.ds(16 * j, 16)
                    rows_v[b, r, sl] = rows_v[b, r, sl] * a
            return rc
        lax.fori_loop(0, CH // 16, grp, 0)
        pltpu.sync_copy(rows_v.at[b], out_sh.at[dstc_v.at[k, 0]], add=True)

    cstart = jnp.where(c == 0, 0, NCH0)
    NQ = jnp.where(c == 0, NCH0 // 4, NCH1 // 4)
    isems = (isem0, isem1, isem2, isem3)
    gsems = (gsem0, gsem1)

    def AB(ci, k):
        wait_idx(ci, k, isems[k])
        fire_g(k, gsems[k % 2])

    for k in range(4):
        fire_idx(cstart + k, k, isems[k])
    AB(cstart, 0)
    AB(cstart + 1, 1)

    def quad(t, carry):
        i0 = cstart + 4 * t
        inr = t + 1 < NQ

        process(0, gsems[0])

        @pl.when(inr)
        def _():
            fire_idx(i0 + 4, 0, isems[0])
        AB(i0 + 2, 2)
        process(1, gsems[1])

        @pl.when(inr)
        def _():
            fire_idx(i0 + 5, 1, isems[1])
        AB(i0 + 3, 3)
        process(2, gsems[0])

        @pl.when(inr)
        def _():
            fire_idx(i0 + 6, 2, isems[2])
            AB(i0 + 4, 0)
        process(3, gsems[1])

        @pl.when(inr)
        def _():
            fire_idx(i0 + 7, 3, isems[3])
            AB(i0 + 5, 1)
        return carry
    lax.fori_loop(0, NQ, quad, 0)

    pltpu.sync_copy(den_v, denp_hbm.at[wid])

    plsc.subcore_barrier()
    pltpu.sync_copy(out_sh.at[pl.ds(s * ROWS_PC, ROWS_PC)],
                    outp_hbm.at[c, pl.ds(s * ROWS_PC, ROWS_PC)])


def _edge_pass(feat, el, er, src3, dst3):
    mesh = plsc.VectorSubcoreMesh(core_axis_name="c", subcore_axis_name="s",
                                  num_cores=NC, num_subcores=NS)
    fn = pl.kernel(
        _edge_body,
        compiler_params=pltpu.CompilerParams(needs_layout_passes=False),
        out_type=[
            jax.ShapeDtypeStruct((NC, NPAD, D), jnp.float32),
            jax.ShapeDtypeStruct((NW, NPAD), jnp.float32),
        ],
        mesh=mesh,
        scratch_types=[
            pltpu.VMEM((NPAD,), jnp.float32),
            pltpu.VMEM((NPAD,), jnp.float32),
            pltpu.VMEM((NPAD,), jnp.float32),
            pltpu.VMEM((4, 1, CH), jnp.int32),
            pltpu.VMEM((4, 1, CH), jnp.int32),
            pltpu.VMEM((2, CH), jnp.float32),
            pltpu.VMEM((2, CH, D), jnp.float32),
            pltpu.MemorySpace.VMEM_SHARED((NPAD, D), jnp.float32),
            pltpu.SemaphoreType.DMA,
            pltpu.SemaphoreType.DMA,
            pltpu.SemaphoreType.DMA,
            pltpu.SemaphoreType.DMA,
            pltpu.SemaphoreType.DMA,
            pltpu.SemaphoreType.DMA,
        ],
    )
    return fn(feat, el, er, src3, dst3)


def _scale_body(outp_hbm, denp_hbm, b_hbm, h_hbm,
                p0_v, p1_v, dd_v, acc_v, b_v):
    c = lax.axis_index("c")
    s = lax.axis_index("s")
    wid = s * NC + c
    R = 128
    NBLK = NPAD // R

    pltpu.sync_copy(b_hbm, b_v)

    def do_block(bid):
        base = pl.multiple_of(bid * R, R)
        bslice = pl.ds(base, R)
        pltpu.sync_copy(outp_hbm.at[0, bslice], p0_v)
        pltpu.sync_copy(outp_hbm.at[1, bslice], p1_v)
        pltpu.sync_copy(denp_hbm.at[:, bslice], dd_v)

        for j in range(R // 16):
            sl = pl.ds(16 * j, 16)
            t = dd_v[0, sl]
            for w in range(1, NW):
                t = t + dd_v[w, sl]
            acc_v[sl] = 1.0 / (t + 1e-9)

        def rgrp(g, rc):
            sc16 = acc_v[pl.ds(pl.multiple_of(g * 16, 16), 16)]
            for r16 in range(16):
                r = g * 16 + r16
                inb = (base + r) < N
                scz = jnp.where(inb, sc16[r16], 0.0)
                for j in range(8):
                    sl = pl.ds(16 * j, 16)
                    hv = jnp.maximum((p0_v[r, sl] + p1_v[r, sl]) * scz
                                     + b_v[pl.ds(16 * j, 16)], 0.0)
                    p0_v[r, sl] = jnp.where(inb, hv, 0.0)
            return rc
        lax.fori_loop(0, R // 16, rgrp, 0)
        pltpu.sync_copy(p0_v, h_hbm.at[bslice])

    do_block(wid)
    do_block(wid + NW)

    @pl.when(wid + 2 * NW < NBLK)
    def _():
        do_block(wid + 2 * NW)


def _scale_pass(outp, denp, brow):
    mesh = plsc.VectorSubcoreMesh(core_axis_name="c", subcore_axis_name="s",
                                  num_cores=NC, num_subcores=NS)
    fn = pl.kernel(
        _scale_body,
        compiler_params=pltpu.CompilerParams(needs_layout_passes=False),
        out_type=jax.ShapeDtypeStruct((NPAD, D), jnp.float32),
        mesh=mesh,
        scratch_types=[
            pltpu.VMEM((128, D), jnp.float32),
            pltpu.VMEM((128, D), jnp.float32),
            pltpu.VMEM((NW, 128), jnp.float32),
            pltpu.VMEM((128,), jnp.float32),
            pltpu.VMEM((D,), jnp.float32),
        ],
    )
    return fn(outp, denp, brow)


def kernel(x, edge_index, W1, al1, ar1, b1, W2, al2, ar2, b2, fcW, fcb):
    f32 = jnp.float32
    x_pad = jnp.zeros((NPAD, D), f32).at[:N].set(x)

    src = edge_index[0]
    dst = edge_index[1]
    pad_e = EPAD - E
    src3 = jnp.concatenate([src, jnp.zeros((pad_e,), jnp.int32)]
                           ).reshape(NS, NCHT, 1, CH)
    dst3 = jnp.concatenate([dst, jnp.full((pad_e,), NPAD - 1, jnp.int32)]
                           ).reshape(NS, NCHT, 1, CH)

    def acols(al, ar):
        a = jnp.zeros((D, D), f32)
        a = a.at[:, 0].set(al[0])
        a = a.at[:, 1].set(ar[0])
        return a

    feat1, elr1 = _mm(x_pad, W1, acols(al1, ar1))
    el1 = elr1[:, 0]
    er1 = elr1[:, 1]
    outp1, denp1 = _edge_pass(feat1, el1, er1, src3, dst3)
    h1 = _scale_pass(outp1, denp1, b1.reshape(D))

    feat2, elr2 = _mm(h1, W2, acols(al2, ar2))
    el2 = elr2[:, 0]
    er2 = elr2[:, 1]
    outp2, denp2 = _edge_pass(feat2, el2, er2, src3, dst3)
    h2 = _scale_pass(outp2, denp2, b2.reshape(D))

    fcw_pad = jnp.zeros((D, D), f32).at[:, :1].set(fcW)
    fcb_pad = jnp.broadcast_to(fcb.reshape(1, 1), (8, D)).astype(f32)
    res = _head(h2, fcw_pad, fcb_pad)
    return res[0:1, 0:1]
